# baseline (speedup 1.0000x reference)
import functools

import jax
import jax.numpy as jnp
from jax import lax
from jax.experimental import pallas as pl
from jax.experimental.pallas import tpu as pltpu
from jax.experimental.pallas import tpu_sc as plsc

_NC = 2
_NS = 16
_NW = _NC * _NS
_CH = 8
_NBUF = 3


def _make_sc_add(B, S, D):
    rows_per_w = S // _NW
    n_chunks = rows_per_w // _CH
    chunk_vecs = _CH * D // 16
    mesh = plsc.VectorSubcoreMesh(core_axis_name="c", subcore_axis_name="s")

    @functools.partial(
        pl.kernel,
        out_type=jax.ShapeDtypeStruct((B, S, D), jnp.float32),
        mesh=mesh,
        scratch_types=[
            [pltpu.VMEM((_CH, D), jnp.float32) for _ in range(_NBUF)],
            [pltpu.VMEM((B, _CH, D), jnp.float32) for _ in range(_NBUF)],
            [pltpu.SemaphoreType.DMA for _ in range(_NBUF)],
            [pltpu.SemaphoreType.DMA for _ in range(_NBUF)],
        ],
    )
    def sc_add(x_hbm, t_hbm, o_hbm, tvs, xvs, in_sems, out_sems):
        wid = lax.axis_index("c") * _NS + lax.axis_index("s")
        row0 = wid * rows_per_w

        def in_copies(g, k):
            r = row0 + g * _CH
            yield pltpu.make_async_copy(
                t_hbm.at[pl.ds(r, _CH), :], tvs[k], in_sems[k])
            yield pltpu.make_async_copy(
                x_hbm.at[:, pl.ds(r, _CH), :], xvs[k], in_sems[k])

        def out_copies(g, k):
            r = row0 + g * _CH
            yield pltpu.make_async_copy(
                xvs[k], o_hbm.at[:, pl.ds(r, _CH), :], out_sems[k])

        def start_in(g, k):
            for c in in_copies(g, k):
                c.start()

        start_in(0, 0)

        n_steps = -(-n_chunks // _NBUF) * _NBUF

        @pl.loop(0, n_steps, step=_NBUF)
        def outer(g0):
            for k in range(_NBUF):
                g = g0 + k
                kn = (k + 1) % _NBUF
                @pl.when(g < n_chunks)
                def _step():
                    @pl.when(g + 1 < n_chunks)
                    def _():
                        @pl.when(g + 1 >= _NBUF)
                        def _():
                            for c in out_copies(g + 1 - _NBUF, kn):
                                c.wait()
                        start_in(g + 1, kn)

                    for c in in_copies(g, k):
                        c.wait()

                    @plsc.parallel_loop(0, D // 16)
                    def add(j):
                        cc = j * 16
                        for r in range(_CH):
                            t = tvs[k][r, pl.ds(cc, 16)]
                            for b in range(B):
                                plsc.addupdate(
                                    xvs[k].at[b, r, pl.ds(cc, 16)], t)

                    for c in out_copies(g, k):
                        c.start()

        for g in range(max(0, n_chunks - _NBUF), n_chunks):
            for c in out_copies(g, g % _NBUF):
                c.wait()

    return sc_add


def kernel(x, pos_table):
    B, S, D = x.shape
    return _make_sc_add(B, S, D)(x, pos_table)


# speedup vs baseline: 1.0072x; 1.0072x over previous
import functools

import jax
import jax.numpy as jnp
from jax import lax
from jax.experimental import pallas as pl
from jax.experimental.pallas import tpu as pltpu
from jax.experimental.pallas import tpu_sc as plsc

_NC = 2
_NS = 16
_NW = _NC * _NS
_CH = 8
_NBUF = 3


def _make_sc_add(B, S, D):
    rows_per_w = S // _NW
    n_chunks = rows_per_w // _CH
    chunk_vecs = _CH * D // 16
    mesh = plsc.VectorSubcoreMesh(core_axis_name="c", subcore_axis_name="s")

    @functools.partial(
        pl.kernel,
        out_type=jax.ShapeDtypeStruct((B, S, D), jnp.float32),
        mesh=mesh,
        scratch_types=[
            [pltpu.VMEM((_CH, D), jnp.float32) for _ in range(_NBUF)],
            [pltpu.VMEM((B, _CH, D), jnp.float32) for _ in range(_NBUF)],
            [pltpu.SemaphoreType.DMA for _ in range(_NBUF)],
            [pltpu.SemaphoreType.DMA for _ in range(_NBUF)],
        ],
    )
    def sc_add(x_hbm, t_hbm, o_hbm, tvs, xvs, in_sems, out_sems):
        wid = lax.axis_index("c") * _NS + lax.axis_index("s")
        row0 = wid * rows_per_w

        def in_copies(g, k):
            r = row0 + g * _CH
            yield pltpu.make_async_copy(
                t_hbm.at[pl.ds(r, _CH), :], tvs[k], in_sems[k])
            yield pltpu.make_async_copy(
                x_hbm.at[:, pl.ds(r, _CH), :], xvs[k], in_sems[k])

        def out_copies(g, k):
            r = row0 + g * _CH
            yield pltpu.make_async_copy(
                xvs[k], o_hbm.at[:, pl.ds(r, _CH), :], out_sems[k])

        def start_in(g, k):
            for c in in_copies(g, k):
                c.start()

        start_in(0, 0)

        n_steps = -(-n_chunks // _NBUF) * _NBUF

        @pl.loop(0, n_steps, step=_NBUF)
        def outer(g0):
            for k in range(_NBUF):
                g = g0 + k
                kn = (k + 1) % _NBUF
                @pl.when(g < n_chunks)
                def _step():
                    @pl.when(g + 1 < n_chunks)
                    def _():
                        @pl.when(g + 1 >= _NBUF)
                        def _():
                            for c in out_copies(g + 1 - _NBUF, kn):
                                c.wait()
                        start_in(g + 1, kn)

                    for c in in_copies(g, k):
                        c.wait()

                    @plsc.parallel_loop(0, chunk_vecs, unroll=4)
                    def add(i):
                        r = i // (D // 16)
                        cc = (i % (D // 16)) * 16
                        t = tvs[k][r, pl.ds(cc, 16)]
                        for b in range(B):
                            plsc.addupdate(xvs[k].at[b, r, pl.ds(cc, 16)], t)

                    for c in out_copies(g, k):
                        c.start()

        for g in range(max(0, n_chunks - _NBUF), n_chunks):
            for c in out_copies(g, g % _NBUF):
                c.wait()

    return sc_add


def kernel(x, pos_table):
    B, S, D = x.shape
    return _make_sc_add(B, S, D)(x, pos_table)
